# baseline (speedup 1.0000x reference)
import functools

import jax
import jax.numpy as jnp
from jax import lax
from jax.experimental import pallas as pl
from jax.experimental.pallas import tpu as pltpu
from jax.experimental.pallas import tpu_sc as plsc

NC = 2
NS = 16
NW = NC * NS
CHUNK = 64
NBUF = 4


def _sc_segment_sums(edge_attr, edge_index, cont_attr, cont_index, num_nodes):
  d = edge_attr.shape[1]
  e = edge_attr.shape[0]
  ec = cont_attr.shape[0]
  nch_e = e // CHUNK
  nch_c = ec // CHUNK
  per_w_e = -(-nch_e // NW)
  per_w_c = -(-nch_c // NW)
  rows_per_tile = -(-num_nodes // (NS * CHUNK)) * CHUNK
  num_nodes = rows_per_tile * NS

  zeros_big = jnp.zeros((CHUNK + 8, d), jnp.float32)
  zeros_hist = jnp.zeros((num_nodes,), jnp.float32)

  mesh = plsc.VectorSubcoreMesh(core_axis_name="c", subcore_axis_name="s")

  @functools.partial(
      pl.kernel,
      out_type=(
          jax.ShapeDtypeStruct((NC, num_nodes, d), jnp.float32),
          jax.ShapeDtypeStruct((NC, NS, num_nodes), jnp.float32),
          jax.ShapeDtypeStruct((NC, num_nodes, d), jnp.float32),
          jax.ShapeDtypeStruct((NC, NS, num_nodes), jnp.float32),
      ),
      mesh=mesh,
      compiler_params=pltpu.CompilerParams(use_tc_tiling_on_sc=False,
                                           needs_layout_passes=False),
      scratch_types=[
          pltpu.VMEM_SHARED((num_nodes, d), jnp.float32),
          pltpu.VMEM((NBUF, CHUNK), jnp.int32),
          pltpu.VMEM((NBUF, CHUNK, d), jnp.float32),
          pltpu.VMEM((num_nodes,), jnp.float32),
          pltpu.SemaphoreType.DMA((NBUF,)),
          pltpu.SemaphoreType.DMA((NBUF,)),
          pltpu.SemaphoreType.DMA((NBUF,)),
          pltpu.SemaphoreType.DMA((NBUF,)),
      ],
  )
  def seg_kernel(eattr, eidx, cattr, cidx, zb_hbm, zh_hbm,
                 msum, mcnt, csum, ccnt,
                 acc, idx_v, rows_v, hist,
                 isem, rsem, ssem, wsem):
    c = lax.axis_index("c")
    s = lax.axis_index("s")
    w = c * NS + s
    row0 = s * rows_per_tile
    nz = rows_per_tile // CHUNK

    pltpu.sync_copy(zh_hbm, hist)

    def zero_acc():
      descs = []
      for k in range(nz):
        r = pl.ds(row0 + k * CHUNK, CHUNK)
        descs.append(pltpu.async_copy(zb_hbm.at[pl.ds(0, CHUNK), :],
                                      acc.at[r, :], wsem.at[0]))
      for dsc in descs:
        dsc.wait()

    def scatter_phase(attr_hbm, idx_hbm, per_w, nch):
      def guard(j, body_fn, guarded):
        if guarded:
          pl.when(j * NW + w < nch)(body_fn)
        else:
          body_fn()

      def issue_load(j, b, guarded=True):
        def go():
          base = (j * NW + w) * CHUNK
          pltpu.async_copy(idx_hbm.at[1, pl.ds(base, CHUNK)], idx_v.at[b],
                           isem.at[b])
          pltpu.async_copy(attr_hbm.at[pl.ds(base, CHUNK), :], rows_v.at[b],
                           rsem.at[b])
        guard(j, go, guarded)

      def wait_load(j, b, guarded=True):
        def go():
          base = (j * NW + w) * CHUNK
          pltpu.make_async_copy(idx_hbm.at[1, pl.ds(base, CHUNK)],
                                idx_v.at[b], isem.at[b]).wait()
          pltpu.make_async_copy(attr_hbm.at[pl.ds(base, CHUNK), :],
                                rows_v.at[b], rsem.at[b]).wait()
        guard(j, go, guarded)

      def start_scat(j, b, guarded=True):
        def go():
          pltpu.async_copy(rows_v.at[b], acc.at[idx_v.at[b]], ssem.at[b],
                           add=True)
        guard(j, go, guarded)

      def hist_update(j, b, guarded=True):
        def go():
          for v in range(CHUNK // 16):
            iv = idx_v[b, pl.ds(v * 16, 16)]
            cnts, lmask = plsc.scan_count(iv)
            plsc.addupdate_scatter(hist, [iv], cnts.astype(jnp.float32),
                                   mask=lmask)
        guard(j, go, guarded)

      def wait_scat(j, b, guarded=True):
        def go():
          pltpu.make_async_copy(rows_v.at[b], acc.at[idx_v.at[b]],
                                ssem.at[b]).wait()
        guard(j, go, guarded)

      for b in range(NBUF):
        issue_load(b, b)

      def make_body(guarded):
        def body(g, carry):
          j0 = g * NBUF
          for b in range(NBUF):
            wait_load(j0 + b, b, guarded)
            start_scat(j0 + b, b, guarded)
            hist_update(j0 + b, b, guarded)
          for b in range(NBUF):
            wait_scat(j0 + b, b, guarded)
            issue_load(j0 + NBUF + b, b, guarded)
          return carry
        return body

      full_j = nch // NW
      g_free = max(0, (full_j - NBUF) // NBUF)
      g_max = -(-per_w // NBUF)
      lax.fori_loop(0, g_free, make_body(False), 0)
      lax.fori_loop(g_free, g_max, make_body(True), 0)

    def dump(sum_out, cnt_out):
      descs = []
      for k in range(nz):
        r = pl.ds(row0 + k * CHUNK, CHUNK)
        descs.append(pltpu.async_copy(acc.at[r, :], sum_out.at[c, r, :],
                                      wsem.at[k % NBUF]))
      descs.append(pltpu.async_copy(hist, cnt_out.at[c, s, :],
                                    wsem.at[0]))
      for dsc in descs:
        dsc.wait()

    zero_acc()
    plsc.subcore_barrier()
    scatter_phase(eattr, eidx, per_w_e, nch_e)
    plsc.subcore_barrier()
    dump(msum, mcnt)
    plsc.subcore_barrier()
    scatter_phase(cattr, cidx, per_w_c, nch_c)
    plsc.subcore_barrier()
    dump(csum, ccnt)

  return seg_kernel(edge_attr, edge_index, cont_attr, cont_index,
                    zeros_big, zeros_hist)


def _mlp(node_attr, msum, mcnt, csum, ccnt, W1, b1, W2, b2, block_n=2048):
  n, d = node_attr.shape

  def mlp_kernel(x_ref, ms_ref, mc_ref, cs_ref, cc_ref,
                 w1_ref, b1_ref, w2_ref, b2_ref, o_ref):
    ms = ms_ref[0] + ms_ref[1]
    ones_w = jnp.ones((NC * NS, 1), jnp.float32)
    mc2 = mc_ref[...].reshape(NC * NS, -1)
    cc2 = cc_ref[...].reshape(NC * NS, -1)
    mc = lax.dot_general(mc2, ones_w, (((0,), (0,)), ((), ())),
                         preferred_element_type=jnp.float32)
    cs = cs_ref[0] + cs_ref[1] - ms
    cc = lax.dot_general(cc2, ones_w, (((0,), (0,)), ((), ())),
                         preferred_element_type=jnp.float32) - mc
    aggm = ms / jnp.maximum(mc, 1.0)
    aggc = cs / jnp.maximum(cc, 1.0)
    x = x_ref[...]
    w1 = w1_ref[...]
    h = (jnp.dot(x, w1[0:d], preferred_element_type=jnp.float32)
         + jnp.dot(aggm, w1[d:2 * d], preferred_element_type=jnp.float32)
         + jnp.dot(aggc, w1[2 * d:3 * d], preferred_element_type=jnp.float32)
         + b1_ref[...])
    h = jnp.maximum(h, 0.0)
    o_ref[...] = (jnp.dot(h, w2_ref[...], preferred_element_type=jnp.float32)
                  + b2_ref[...])

  return pl.pallas_call(
      mlp_kernel,
      grid=(-(-n // block_n),),
      in_specs=[
          pl.BlockSpec((block_n, d), lambda i: (i, 0)),
          pl.BlockSpec((NC, block_n, d), lambda i: (0, i, 0)),
          pl.BlockSpec((NC, NS, block_n), lambda i: (0, 0, i)),
          pl.BlockSpec((NC, block_n, d), lambda i: (0, i, 0)),
          pl.BlockSpec((NC, NS, block_n), lambda i: (0, 0, i)),
          pl.BlockSpec((3 * d, d), lambda i: (0, 0)),
          pl.BlockSpec((1, d), lambda i: (0, 0)),
          pl.BlockSpec((d, d), lambda i: (0, 0)),
          pl.BlockSpec((1, d), lambda i: (0, 0)),
      ],
      out_specs=pl.BlockSpec((block_n, d), lambda i: (i, 0)),
      out_shape=jax.ShapeDtypeStruct((n, d), jnp.float32),
  )(node_attr, msum, mcnt, csum, ccnt,
    W1, b1.reshape(1, d), W2, b2.reshape(1, d))


def kernel(node_attr, edge_attr, edge_index, edge_contact_attr,
           edge_contact_index, W1, b1, W2, b2):
  num_nodes = node_attr.shape[0]
  msum, mcnt, csum, ccnt = _sc_segment_sums(
      edge_attr, edge_index, edge_contact_attr, edge_contact_index, num_nodes)
  return _mlp(node_attr, msum, mcnt, csum, ccnt, W1, b1, W2, b2)


# speedup vs baseline: 1.0611x; 1.0611x over previous
import functools

import jax
import jax.numpy as jnp
from jax import lax
from jax.experimental import pallas as pl
from jax.experimental.pallas import tpu as pltpu
from jax.experimental.pallas import tpu_sc as plsc

NC = 2
NS = 16
NW = NC * NS
CHUNK = 64
NBUF = 4


def _sc_segment_sums(edge_attr, edge_index, cont_attr, cont_index, num_nodes):
  d = edge_attr.shape[1]
  e = edge_attr.shape[0]
  ec = cont_attr.shape[0]
  nch_e = e // CHUNK
  nch_c = ec // CHUNK
  per_w_e = -(-nch_e // NW)
  per_w_c = -(-nch_c // NW)
  rows_per_tile = -(-num_nodes // (NS * CHUNK)) * CHUNK
  num_nodes = rows_per_tile * NS

  zeros_big = jnp.zeros((CHUNK + 8, d), jnp.float32)
  zeros_hist = jnp.zeros((num_nodes,), jnp.float32)

  mesh = plsc.VectorSubcoreMesh(core_axis_name="c", subcore_axis_name="s")

  @functools.partial(
      pl.kernel,
      out_type=(
          jax.ShapeDtypeStruct((NC, num_nodes, d), jnp.float32),
          jax.ShapeDtypeStruct((NC, NS, num_nodes), jnp.float32),
          jax.ShapeDtypeStruct((NC, num_nodes, d), jnp.float32),
          jax.ShapeDtypeStruct((NC, NS, num_nodes), jnp.float32),
      ),
      mesh=mesh,
      compiler_params=pltpu.CompilerParams(use_tc_tiling_on_sc=False,
                                           needs_layout_passes=False),
      scratch_types=[
          pltpu.VMEM_SHARED((num_nodes, d), jnp.float32),
          pltpu.VMEM((NBUF, CHUNK), jnp.int32),
          pltpu.VMEM((NBUF, CHUNK, d), jnp.float32),
          pltpu.VMEM((num_nodes,), jnp.float32),
          pltpu.SemaphoreType.DMA((NBUF,)),
          pltpu.SemaphoreType.DMA((NBUF,)),
          pltpu.SemaphoreType.DMA((NBUF,)),
          pltpu.SemaphoreType.DMA((NBUF,)),
      ],
  )
  def seg_kernel(eattr, eidx, cattr, cidx, zb_hbm, zh_hbm,
                 msum, mcnt, csum, ccnt,
                 acc, idx_v, rows_v, hist,
                 isem, rsem, ssem, wsem):
    c = lax.axis_index("c")
    s = lax.axis_index("s")
    w = c * NS + s
    row0 = s * rows_per_tile
    nz = rows_per_tile // CHUNK

    pltpu.sync_copy(zh_hbm, hist)

    def zero_acc():
      pltpu.sync_copy(zb_hbm.at[pl.ds(0, CHUNK), :], rows_v.at[0])
      descs = []
      for k in range(nz):
        r = pl.ds(row0 + k * CHUNK, CHUNK)
        descs.append(pltpu.async_copy(rows_v.at[0], acc.at[r, :],
                                      wsem.at[0]))
      for dsc in descs:
        dsc.wait()

    def scatter_phase(attr_hbm, idx_hbm, per_w, nch):
      def guard(j, body_fn, guarded):
        if guarded:
          pl.when(j * NW + w < nch)(body_fn)
        else:
          body_fn()

      def issue_load(j, b, guarded=True):
        def go():
          base = (j * NW + w) * CHUNK
          pltpu.async_copy(idx_hbm.at[1, pl.ds(base, CHUNK)], idx_v.at[b],
                           isem.at[b])
          pltpu.async_copy(attr_hbm.at[pl.ds(base, CHUNK), :], rows_v.at[b],
                           rsem.at[b])
        guard(j, go, guarded)

      def wait_load(j, b, guarded=True):
        def go():
          base = (j * NW + w) * CHUNK
          pltpu.make_async_copy(idx_hbm.at[1, pl.ds(base, CHUNK)],
                                idx_v.at[b], isem.at[b]).wait()
          pltpu.make_async_copy(attr_hbm.at[pl.ds(base, CHUNK), :],
                                rows_v.at[b], rsem.at[b]).wait()
        guard(j, go, guarded)

      def start_scat(j, b, guarded=True):
        def go():
          pltpu.async_copy(rows_v.at[b], acc.at[idx_v.at[b]], ssem.at[b],
                           add=True)
        guard(j, go, guarded)

      def hist_update(j, b, guarded=True):
        def go():
          for v in range(CHUNK // 16):
            iv = idx_v[b, pl.ds(v * 16, 16)]
            cnts, lmask = plsc.scan_count(iv)
            plsc.addupdate_scatter(hist, [iv], cnts.astype(jnp.float32),
                                   mask=lmask)
        guard(j, go, guarded)

      def wait_scat(j, b, guarded=True):
        def go():
          pltpu.make_async_copy(rows_v.at[b], acc.at[idx_v.at[b]],
                                ssem.at[b]).wait()
        guard(j, go, guarded)

      for b in range(NBUF):
        issue_load(b, b)

      def make_body(guarded):
        def body(g, carry):
          j0 = g * NBUF
          for b in range(NBUF):
            wait_load(j0 + b, b, guarded)
            start_scat(j0 + b, b, guarded)
            hist_update(j0 + b, b, guarded)
          for b in range(NBUF):
            wait_scat(j0 + b, b, guarded)
            issue_load(j0 + NBUF + b, b, guarded)
          return carry
        return body

      full_j = nch // NW
      g_free = max(0, (full_j - NBUF) // NBUF)
      g_max = -(-per_w // NBUF)
      lax.fori_loop(0, g_free, make_body(False), 0)
      lax.fori_loop(g_free, g_max, make_body(True), 0)

    def dump(sum_out, cnt_out):
      descs = [None] * NBUF
      for k in range(nz):
        b = k % NBUF
        r = pl.ds(row0 + k * CHUNK, CHUNK)
        if descs[b] is not None:
          descs[b].wait()
        pltpu.sync_copy(acc.at[r, :], rows_v.at[b])
        descs[b] = pltpu.async_copy(rows_v.at[b], sum_out.at[c, r, :],
                                    wsem.at[b])
      for dsc in descs:
        if dsc is not None:
          dsc.wait()
      pltpu.sync_copy(hist, cnt_out.at[c, s, :])

    zero_acc()
    plsc.subcore_barrier()
    scatter_phase(eattr, eidx, per_w_e, nch_e)
    plsc.subcore_barrier()
    dump(msum, mcnt)
    plsc.subcore_barrier()
    scatter_phase(cattr, cidx, per_w_c, nch_c)
    plsc.subcore_barrier()
    dump(csum, ccnt)

  return seg_kernel(edge_attr, edge_index, cont_attr, cont_index,
                    zeros_big, zeros_hist)


def _mlp(node_attr, msum, mcnt, csum, ccnt, W1, b1, W2, b2, block_n=2048):
  n, d = node_attr.shape

  def mlp_kernel(x_ref, ms_ref, mc_ref, cs_ref, cc_ref,
                 w1_ref, b1_ref, w2_ref, b2_ref, o_ref):
    ms = ms_ref[0] + ms_ref[1]
    ones_w = jnp.ones((NC * NS, 1), jnp.float32)
    mc2 = mc_ref[...].reshape(NC * NS, -1)
    cc2 = cc_ref[...].reshape(NC * NS, -1)
    mc = lax.dot_general(mc2, ones_w, (((0,), (0,)), ((), ())),
                         preferred_element_type=jnp.float32)
    cs = cs_ref[0] + cs_ref[1] - ms
    cc = lax.dot_general(cc2, ones_w, (((0,), (0,)), ((), ())),
                         preferred_element_type=jnp.float32) - mc
    aggm = ms / jnp.maximum(mc, 1.0)
    aggc = cs / jnp.maximum(cc, 1.0)
    x = x_ref[...]
    w1 = w1_ref[...]
    h = (jnp.dot(x, w1[0:d], preferred_element_type=jnp.float32)
         + jnp.dot(aggm, w1[d:2 * d], preferred_element_type=jnp.float32)
         + jnp.dot(aggc, w1[2 * d:3 * d], preferred_element_type=jnp.float32)
         + b1_ref[...])
    h = jnp.maximum(h, 0.0)
    o_ref[...] = (jnp.dot(h, w2_ref[...], preferred_element_type=jnp.float32)
                  + b2_ref[...])

  return pl.pallas_call(
      mlp_kernel,
      grid=(-(-n // block_n),),
      in_specs=[
          pl.BlockSpec((block_n, d), lambda i: (i, 0)),
          pl.BlockSpec((NC, block_n, d), lambda i: (0, i, 0)),
          pl.BlockSpec((NC, NS, block_n), lambda i: (0, 0, i)),
          pl.BlockSpec((NC, block_n, d), lambda i: (0, i, 0)),
          pl.BlockSpec((NC, NS, block_n), lambda i: (0, 0, i)),
          pl.BlockSpec((3 * d, d), lambda i: (0, 0)),
          pl.BlockSpec((1, d), lambda i: (0, 0)),
          pl.BlockSpec((d, d), lambda i: (0, 0)),
          pl.BlockSpec((1, d), lambda i: (0, 0)),
      ],
      out_specs=pl.BlockSpec((block_n, d), lambda i: (i, 0)),
      out_shape=jax.ShapeDtypeStruct((n, d), jnp.float32),
  )(node_attr, msum, mcnt, csum, ccnt,
    W1, b1.reshape(1, d), W2, b2.reshape(1, d))


def kernel(node_attr, edge_attr, edge_index, edge_contact_attr,
           edge_contact_index, W1, b1, W2, b2):
  num_nodes = node_attr.shape[0]
  msum, mcnt, csum, ccnt = _sc_segment_sums(
      edge_attr, edge_index, edge_contact_attr, edge_contact_index, num_nodes)
  return _mlp(node_attr, msum, mcnt, csum, ccnt, W1, b1, W2, b2)


# speedup vs baseline: 1.0751x; 1.0132x over previous
import functools

import jax
import jax.numpy as jnp
from jax import lax
from jax.experimental import pallas as pl
from jax.experimental.pallas import tpu as pltpu
from jax.experimental.pallas import tpu_sc as plsc

NC = 2
NS = 16
NW = NC * NS
CHUNK = 64
NBUF = 4


def _sc_segment_sums(edge_attr, edge_index, cont_attr, cont_index, num_nodes):
  d = edge_attr.shape[1]
  e = edge_attr.shape[0]
  ec = cont_attr.shape[0]
  nch_e = e // CHUNK
  nch_c = ec // CHUNK
  per_w_e = -(-nch_e // NW)
  per_w_c = -(-nch_c // NW)
  rows_per_tile = -(-num_nodes // (NS * CHUNK)) * CHUNK
  num_nodes = rows_per_tile * NS

  mesh = plsc.VectorSubcoreMesh(core_axis_name="c", subcore_axis_name="s")

  @functools.partial(
      pl.kernel,
      out_type=(
          jax.ShapeDtypeStruct((NC, num_nodes, d), jnp.float32),
          jax.ShapeDtypeStruct((NC, NS, num_nodes), jnp.float32),
          jax.ShapeDtypeStruct((NC, num_nodes, d), jnp.float32),
          jax.ShapeDtypeStruct((NC, NS, num_nodes), jnp.float32),
      ),
      mesh=mesh,
      compiler_params=pltpu.CompilerParams(use_tc_tiling_on_sc=False,
                                           needs_layout_passes=False),
      scratch_types=[
          pltpu.VMEM_SHARED((num_nodes, d), jnp.float32),
          pltpu.VMEM((NBUF, CHUNK), jnp.int32),
          pltpu.VMEM((NBUF, CHUNK, d), jnp.float32),
          pltpu.VMEM((num_nodes,), jnp.float32),
          pltpu.SemaphoreType.DMA((NBUF,)),
          pltpu.SemaphoreType.DMA((NBUF,)),
          pltpu.SemaphoreType.DMA((NBUF,)),
          pltpu.SemaphoreType.DMA((NBUF,)),
      ],
  )
  def seg_kernel(eattr, eidx, cattr, cidx,
                 msum, mcnt, csum, ccnt,
                 acc, idx_v, rows_v, hist,
                 isem, rsem, ssem, wsem):
    c = lax.axis_index("c")
    s = lax.axis_index("s")
    w = c * NS + s
    row0 = s * rows_per_tile
    nz = rows_per_tile // CHUNK
    z16 = jnp.zeros((16,), jnp.float32)

    def zh_body(i, carry):
      hist[pl.ds(i * 16, 16)] = z16
      return carry

    lax.fori_loop(0, num_nodes // 16, zh_body, 0)

    def zero_acc():
      def zr_body(r, carry):
        for vcol in range(d // 16):
          rows_v[0, r, pl.ds(vcol * 16, 16)] = z16
        return carry

      lax.fori_loop(0, CHUNK, zr_body, 0)
      descs = []
      for k in range(nz):
        r = pl.ds(row0 + k * CHUNK, CHUNK)
        descs.append(pltpu.async_copy(rows_v.at[0], acc.at[r, :],
                                      wsem.at[0]))
      for dsc in descs:
        dsc.wait()

    def scatter_phase(attr_hbm, idx_hbm, per_w, nch):
      def guard(j, body_fn, guarded):
        if guarded:
          pl.when(j * NW + w < nch)(body_fn)
        else:
          body_fn()

      def issue_load(j, b, guarded=True):
        def go():
          base = (j * NW + w) * CHUNK
          pltpu.async_copy(idx_hbm.at[1, pl.ds(base, CHUNK)], idx_v.at[b],
                           isem.at[b])
          pltpu.async_copy(attr_hbm.at[pl.ds(base, CHUNK), :], rows_v.at[b],
                           rsem.at[b])
        guard(j, go, guarded)

      def wait_load(j, b, guarded=True):
        def go():
          base = (j * NW + w) * CHUNK
          pltpu.make_async_copy(idx_hbm.at[1, pl.ds(base, CHUNK)],
                                idx_v.at[b], isem.at[b]).wait()
          pltpu.make_async_copy(attr_hbm.at[pl.ds(base, CHUNK), :],
                                rows_v.at[b], rsem.at[b]).wait()
        guard(j, go, guarded)

      def start_scat(j, b, guarded=True):
        def go():
          pltpu.async_copy(rows_v.at[b], acc.at[idx_v.at[b]], ssem.at[b],
                           add=True)
        guard(j, go, guarded)

      def hist_update(j, b, guarded=True):
        def go():
          for v in range(CHUNK // 16):
            iv = idx_v[b, pl.ds(v * 16, 16)]
            cnts, lmask = plsc.scan_count(iv)
            plsc.addupdate_scatter(hist, [iv], cnts.astype(jnp.float32),
                                   mask=lmask)
        guard(j, go, guarded)

      def wait_scat(j, b, guarded=True):
        def go():
          pltpu.make_async_copy(rows_v.at[b], acc.at[idx_v.at[b]],
                                ssem.at[b]).wait()
        guard(j, go, guarded)

      for b in range(NBUF):
        issue_load(b, b)

      def make_body(guarded):
        def body(g, carry):
          j0 = g * NBUF
          for b in range(NBUF):
            wait_load(j0 + b, b, guarded)
            start_scat(j0 + b, b, guarded)
            hist_update(j0 + b, b, guarded)
          for b in range(NBUF):
            wait_scat(j0 + b, b, guarded)
            issue_load(j0 + NBUF + b, b, guarded)
          return carry
        return body

      full_j = nch // NW
      g_free = max(0, (full_j - NBUF) // NBUF)
      g_max = -(-per_w // NBUF)
      lax.fori_loop(0, g_free, make_body(False), 0)
      lax.fori_loop(g_free, g_max, make_body(True), 0)

    def dump(sum_out, cnt_out):
      descs = [None] * NBUF
      for k in range(nz):
        b = k % NBUF
        r = pl.ds(row0 + k * CHUNK, CHUNK)
        if descs[b] is not None:
          descs[b].wait()
        pltpu.sync_copy(acc.at[r, :], rows_v.at[b])
        descs[b] = pltpu.async_copy(rows_v.at[b], sum_out.at[c, r, :],
                                    wsem.at[b])
      for dsc in descs:
        if dsc is not None:
          dsc.wait()
      pltpu.sync_copy(hist, cnt_out.at[c, s, :])

    zero_acc()
    plsc.subcore_barrier()
    scatter_phase(eattr, eidx, per_w_e, nch_e)
    plsc.subcore_barrier()
    dump(msum, mcnt)
    plsc.subcore_barrier()
    scatter_phase(cattr, cidx, per_w_c, nch_c)
    plsc.subcore_barrier()
    dump(csum, ccnt)

  return seg_kernel(edge_attr, edge_index, cont_attr, cont_index)


def _mlp(node_attr, msum, mcnt, csum, ccnt, W1, b1, W2, b2, block_n=2048):
  n, d = node_attr.shape

  def mlp_kernel(x_ref, ms_ref, mc_ref, cs_ref, cc_ref,
                 w1_ref, b1_ref, w2_ref, b2_ref, o_ref):
    ms = ms_ref[0] + ms_ref[1]
    ones_w = jnp.ones((NC * NS, 1), jnp.float32)
    mc2 = mc_ref[...].reshape(NC * NS, -1)
    cc2 = cc_ref[...].reshape(NC * NS, -1)
    mc = lax.dot_general(mc2, ones_w, (((0,), (0,)), ((), ())),
                         preferred_element_type=jnp.float32)
    cs = cs_ref[0] + cs_ref[1] - ms
    cc = lax.dot_general(cc2, ones_w, (((0,), (0,)), ((), ())),
                         preferred_element_type=jnp.float32) - mc
    aggm = ms / jnp.maximum(mc, 1.0)
    aggc = cs / jnp.maximum(cc, 1.0)
    x = x_ref[...]
    w1 = w1_ref[...]
    h = (jnp.dot(x, w1[0:d], preferred_element_type=jnp.float32)
         + jnp.dot(aggm, w1[d:2 * d], preferred_element_type=jnp.float32)
         + jnp.dot(aggc, w1[2 * d:3 * d], preferred_element_type=jnp.float32)
         + b1_ref[...])
    h = jnp.maximum(h, 0.0)
    o_ref[...] = (jnp.dot(h, w2_ref[...], preferred_element_type=jnp.float32)
                  + b2_ref[...])

  return pl.pallas_call(
      mlp_kernel,
      grid=(-(-n // block_n),),
      in_specs=[
          pl.BlockSpec((block_n, d), lambda i: (i, 0)),
          pl.BlockSpec((NC, block_n, d), lambda i: (0, i, 0)),
          pl.BlockSpec((NC, NS, block_n), lambda i: (0, 0, i)),
          pl.BlockSpec((NC, block_n, d), lambda i: (0, i, 0)),
          pl.BlockSpec((NC, NS, block_n), lambda i: (0, 0, i)),
          pl.BlockSpec((3 * d, d), lambda i: (0, 0)),
          pl.BlockSpec((1, d), lambda i: (0, 0)),
          pl.BlockSpec((d, d), lambda i: (0, 0)),
          pl.BlockSpec((1, d), lambda i: (0, 0)),
      ],
      out_specs=pl.BlockSpec((block_n, d), lambda i: (i, 0)),
      out_shape=jax.ShapeDtypeStruct((n, d), jnp.float32),
  )(node_attr, msum, mcnt, csum, ccnt,
    W1, b1.reshape(1, d), W2, b2.reshape(1, d))


def kernel(node_attr, edge_attr, edge_index, edge_contact_attr,
           edge_contact_index, W1, b1, W2, b2):
  num_nodes = node_attr.shape[0]
  msum, mcnt, csum, ccnt = _sc_segment_sums(
      edge_attr, edge_index, edge_contact_attr, edge_contact_index, num_nodes)
  return _mlp(node_attr, msum, mcnt, csum, ccnt, W1, b1, W2, b2)


# speedup vs baseline: 1.0777x; 1.0024x over previous
import functools

import jax
import jax.numpy as jnp
from jax import lax
from jax.experimental import pallas as pl
from jax.experimental.pallas import tpu as pltpu
from jax.experimental.pallas import tpu_sc as plsc

NC = 2
NS = 16
NW = NC * NS
CHUNK = 64
NBUF = 4


def _sc_segment_sums(edge_attr, edge_index, cont_attr, cont_index, num_nodes):
  d = edge_attr.shape[1]
  e = edge_attr.shape[0]
  ec = cont_attr.shape[0]
  nch_e = e // CHUNK
  nch_c = ec // CHUNK
  per_w_e = -(-nch_e // NW)
  per_w_c = -(-nch_c // NW)
  rows_per_tile = -(-num_nodes // (NS * CHUNK)) * CHUNK
  num_nodes = rows_per_tile * NS

  mesh = plsc.VectorSubcoreMesh(core_axis_name="c", subcore_axis_name="s")

  @functools.partial(
      pl.kernel,
      out_type=(
          jax.ShapeDtypeStruct((NC, num_nodes, d), jnp.float32),
          jax.ShapeDtypeStruct((NC, NS, num_nodes), jnp.float32),
          jax.ShapeDtypeStruct((NC, num_nodes, d), jnp.float32),
          jax.ShapeDtypeStruct((NC, NS, num_nodes), jnp.float32),
      ),
      mesh=mesh,
      compiler_params=pltpu.CompilerParams(use_tc_tiling_on_sc=False,
                                           needs_layout_passes=False),
      scratch_types=[
          pltpu.VMEM_SHARED((num_nodes, d), jnp.float32),
          pltpu.VMEM((NBUF, CHUNK), jnp.int32),
          pltpu.VMEM((NBUF, CHUNK, d), jnp.float32),
          pltpu.VMEM((num_nodes,), jnp.float32),
          pltpu.SemaphoreType.DMA((NBUF,)),
          pltpu.SemaphoreType.DMA((NBUF,)),
          pltpu.SemaphoreType.DMA((NBUF,)),
          pltpu.SemaphoreType.DMA((NBUF,)),
      ],
  )
  def seg_kernel(eattr, eidx, cattr, cidx,
                 msum, mcnt, csum, ccnt,
                 acc, idx_v, rows_v, hist,
                 isem, rsem, ssem, wsem):
    c = lax.axis_index("c")
    s = lax.axis_index("s")
    w = c * NS + s
    row0 = s * rows_per_tile
    nz = rows_per_tile // CHUNK
    z16 = jnp.zeros((16,), jnp.float32)

    def zh_body(i, carry):
      hist[pl.ds(i * 16, 16)] = z16
      return carry

    lax.fori_loop(0, num_nodes // 16, zh_body, 0)

    def zero_acc():
      def zr_body(r, carry):
        for vcol in range(d // 16):
          rows_v[0, r, pl.ds(vcol * 16, 16)] = z16
        return carry

      lax.fori_loop(0, CHUNK, zr_body, 0)
      descs = []
      for k in range(nz):
        r = pl.ds(row0 + k * CHUNK, CHUNK)
        descs.append(pltpu.async_copy(rows_v.at[0], acc.at[r, :],
                                      wsem.at[0]))
      for dsc in descs:
        dsc.wait()

    def scatter_phase(attr_hbm, idx_hbm, per_w, nch):
      def guard(j, body_fn, guarded):
        if guarded:
          pl.when(j * NW + w < nch)(body_fn)
        else:
          body_fn()

      def issue_load(j, b, guarded=True):
        def go():
          base = (j * NW + w) * CHUNK
          pltpu.async_copy(idx_hbm.at[1, pl.ds(base, CHUNK)], idx_v.at[b],
                           isem.at[b])
          pltpu.async_copy(attr_hbm.at[pl.ds(base, CHUNK), :], rows_v.at[b],
                           rsem.at[b])
        guard(j, go, guarded)

      def wait_load(j, b, guarded=True):
        def go():
          base = (j * NW + w) * CHUNK
          pltpu.make_async_copy(idx_hbm.at[1, pl.ds(base, CHUNK)],
                                idx_v.at[b], isem.at[b]).wait()
          pltpu.make_async_copy(attr_hbm.at[pl.ds(base, CHUNK), :],
                                rows_v.at[b], rsem.at[b]).wait()
        guard(j, go, guarded)

      def start_scat(j, b, guarded=True):
        def go():
          pltpu.async_copy(rows_v.at[b], acc.at[idx_v.at[b]], ssem.at[b],
                           add=True)
        guard(j, go, guarded)

      def hist_update(j, b, guarded=True):
        def go():
          for v in range(CHUNK // 16):
            iv = idx_v[b, pl.ds(v * 16, 16)]
            cnts, lmask = plsc.scan_count(iv)
            plsc.addupdate_scatter(hist, [iv], cnts.astype(jnp.float32),
                                   mask=lmask)
        guard(j, go, guarded)

      def wait_scat(j, b, guarded=True):
        def go():
          pltpu.make_async_copy(rows_v.at[b], acc.at[idx_v.at[b]],
                                ssem.at[b]).wait()
        guard(j, go, guarded)

      for b in range(NBUF):
        issue_load(b, b)

      def make_body(guarded):
        def body(g, carry):
          j0 = g * NBUF
          for b in range(NBUF):
            wait_load(j0 + b, b, guarded)
            start_scat(j0 + b, b, guarded)
            hist_update(j0 + b, b, guarded)
          for b in range(NBUF):
            wait_scat(j0 + b, b, guarded)
            issue_load(j0 + NBUF + b, b, guarded)
          return carry
        return body

      full_j = nch // NW
      g_free = max(0, (full_j - NBUF) // NBUF)
      g_max = -(-per_w // NBUF)
      lax.fori_loop(0, g_free, make_body(False), 0)
      lax.fori_loop(g_free, g_max, make_body(True), 0)

    def dump(sum_out, cnt_out):
      hdesc = pltpu.async_copy(hist, cnt_out.at[c, s, :], rsem.at[0])
      descs = [None] * NBUF
      for k in range(nz):
        b = k % NBUF
        r = pl.ds(row0 + k * CHUNK, CHUNK)
        if descs[b] is not None:
          descs[b].wait()
        pltpu.sync_copy(acc.at[r, :], rows_v.at[b])
        descs[b] = pltpu.async_copy(rows_v.at[b], sum_out.at[c, r, :],
                                    wsem.at[b])
      for dsc in descs:
        if dsc is not None:
          dsc.wait()
      hdesc.wait()

    zero_acc()
    plsc.subcore_barrier()
    scatter_phase(eattr, eidx, per_w_e, nch_e)
    plsc.subcore_barrier()
    dump(msum, mcnt)
    plsc.subcore_barrier()
    scatter_phase(cattr, cidx, per_w_c, nch_c)
    plsc.subcore_barrier()
    dump(csum, ccnt)

  return seg_kernel(edge_attr, edge_index, cont_attr, cont_index)


def _mlp(node_attr, msum, mcnt, csum, ccnt, W1, b1, W2, b2, block_n=2048):
  n, d = node_attr.shape

  def mlp_kernel(x_ref, ms_ref, mc_ref, cs_ref, cc_ref,
                 w1_ref, b1_ref, w2_ref, b2_ref, o_ref):
    ms = ms_ref[0] + ms_ref[1]
    ones_w = jnp.ones((NC * NS, 1), jnp.float32)
    mc2 = mc_ref[...].reshape(NC * NS, -1)
    cc2 = cc_ref[...].reshape(NC * NS, -1)
    mc = lax.dot_general(mc2, ones_w, (((0,), (0,)), ((), ())),
                         preferred_element_type=jnp.float32)
    cs = cs_ref[0] + cs_ref[1] - ms
    cc = lax.dot_general(cc2, ones_w, (((0,), (0,)), ((), ())),
                         preferred_element_type=jnp.float32) - mc
    aggm = ms / jnp.maximum(mc, 1.0)
    aggc = cs / jnp.maximum(cc, 1.0)
    x = x_ref[...]
    w1 = w1_ref[...]
    h = (jnp.dot(x, w1[0:d], preferred_element_type=jnp.float32)
         + jnp.dot(aggm, w1[d:2 * d], preferred_element_type=jnp.float32)
         + jnp.dot(aggc, w1[2 * d:3 * d], preferred_element_type=jnp.float32)
         + b1_ref[...])
    h = jnp.maximum(h, 0.0)
    o_ref[...] = (jnp.dot(h, w2_ref[...], preferred_element_type=jnp.float32)
                  + b2_ref[...])

  return pl.pallas_call(
      mlp_kernel,
      grid=(-(-n // block_n),),
      in_specs=[
          pl.BlockSpec((block_n, d), lambda i: (i, 0)),
          pl.BlockSpec((NC, block_n, d), lambda i: (0, i, 0)),
          pl.BlockSpec((NC, NS, block_n), lambda i: (0, 0, i)),
          pl.BlockSpec((NC, block_n, d), lambda i: (0, i, 0)),
          pl.BlockSpec((NC, NS, block_n), lambda i: (0, 0, i)),
          pl.BlockSpec((3 * d, d), lambda i: (0, 0)),
          pl.BlockSpec((1, d), lambda i: (0, 0)),
          pl.BlockSpec((d, d), lambda i: (0, 0)),
          pl.BlockSpec((1, d), lambda i: (0, 0)),
      ],
      out_specs=pl.BlockSpec((block_n, d), lambda i: (i, 0)),
      out_shape=jax.ShapeDtypeStruct((n, d), jnp.float32),
  )(node_attr, msum, mcnt, csum, ccnt,
    W1, b1.reshape(1, d), W2, b2.reshape(1, d))


def kernel(node_attr, edge_attr, edge_index, edge_contact_attr,
           edge_contact_index, W1, b1, W2, b2):
  num_nodes = node_attr.shape[0]
  msum, mcnt, csum, ccnt = _sc_segment_sums(
      edge_attr, edge_index, edge_contact_attr, edge_contact_index, num_nodes)
  return _mlp(node_attr, msum, mcnt, csum, ccnt, W1, b1, W2, b2)
